# baseline (speedup 1.0000x reference)
import functools

import jax
import jax.numpy as jnp
from jax import lax
from jax.experimental import pallas as pl
from jax.experimental.pallas import tpu as pltpu
from jax.experimental.pallas import tpu_sc as plsc

B = 8
N = 512 * 512
NC, NS, L = 2, 16, 16
NW = NC * NS
E = (B * N) // NW
CHUNK = 8192
NCHUNK = E // CHUNK
GROUPS = B // NC
WPG = NS // GROUPS
SROW = 3 * L


def _sel4(g, v0, v1, v2, v3):
    return jnp.where(g == 0, v0, jnp.where(g == 1, v1, jnp.where(g == 2, v2, v3)))


def _ohem_body(o_hbm, t_hbm, out_hbm,
               o_buf, t_buf, mbuf,
               stat_stage, stat_sh, stat_rd,
               icnt_stage, icnt_sh, icnt_rd,
               gsum_stage, gsum_sh, gsum_rd,
               row_buf, dsem0, dsem1, dsem2):
    c = lax.axis_index("c")
    s = lax.axis_index("s")
    g = s // WPG
    s_id = c * GROUPS + g
    row0 = (s % WPG) * (E // 512)
    CROWS = CHUNK // 512

    sems = (dsem0, dsem1, dsem2)
    NBUF = 3

    def start(ci):
        b = ci % NBUF
        src = (s_id, pl.ds(row0 + ci * CROWS, CROWS), slice(None))
        ho = pltpu.async_copy(o_hbm.at[src], o_buf.at[b], sems[b])
        ht = pltpu.async_copy(t_hbm.at[src], t_buf.at[b], sems[b])
        return ho, ht

    acc = tuple(jnp.zeros((L,), jnp.float32) for _ in range(3))
    handles = [start(0), start(1), None]
    for ci in range(NCHUNK):
        b = ci % NBUF
        if ci + 2 < NCHUNK:
            handles[(ci + 2) % NBUF] = start(ci + 2)
        ho, ht = handles[b]
        ho.wait()
        ht.wait()

        def chunk_body(i, accs, ci=ci, b=b):
            cntf, spos, sall = accs
            r = i // 32
            col = (i % 32) * L
            ov = o_buf[b, r, pl.ds(col, L)]
            tv = t_buf[b, r, pl.ds(col, L)]
            d = ov - tv
            lv = d * d
            pos = tv > 0.0
            bits = lax.bitcast_convert_type(lv, jnp.int32)
            mbuf[pl.ds(ci * CHUNK + i * L, L)] = jnp.where(
                pos, jnp.int32(-1), bits)
            return (cntf + jnp.where(pos, 1.0, 0.0),
                    spos + jnp.where(pos, lv, 0.0),
                    sall + lv)

        acc = plsc.parallel_loop(0, CHUNK // L, step=1, unroll=8,
                                 carry=acc)(chunk_body)

    cntf, spos, sall = acc

    stat_stage[pl.ds(0, L)] = cntf
    stat_stage[pl.ds(L, L)] = spos
    stat_stage[pl.ds(2 * L, L)] = sall
    pltpu.sync_copy(stat_stage, stat_sh.at[pl.ds(s * SROW, SROW)])
    plsc.subcore_barrier()
    pltpu.sync_copy(stat_sh, stat_rd)

    ps, sps, sas, ks, negs = [], [], [], [], []
    for gi in range(GROUPS):
        cv = jnp.zeros((L,), jnp.float32)
        sv = jnp.zeros((L,), jnp.float32)
        av = jnp.zeros((L,), jnp.float32)
        for wi in range(WPG):
            r = (gi * WPG + wi) * SROW
            cv = cv + stat_rd[pl.ds(r, L)]
            sv = sv + stat_rd[pl.ds(r + L, L)]
            av = av + stat_rd[pl.ds(r + 2 * L, L)]
        p_i = jnp.sum(cv).astype(jnp.int32)
        neg_i = N - p_i
        k0 = 3 * p_i
        k = jnp.where(k0 + p_i > N, neg_i, k0)
        ps.append(p_i)
        sps.append(jnp.sum(sv))
        sas.append(jnp.sum(av))
        ks.append(k)
        negs.append(neg_i)

    def bc(x):
        return jnp.broadcast_to(x, (L,))

    fasts, needs, pms = [], [], []
    nf = jnp.float32(N)
    for gi in range(GROUPS):
        pf = ps[gi].astype(jnp.float32)
        negf = negs[gi].astype(jnp.float32)
        pm = bc(sps[gi]) / bc(pf)
        mean_all = bc(sas[gi]) / bc(nf)
        fast = jnp.where(ks[gi] < 10, mean_all,
                         pm + bc(sas[gi] - sps[gi]) / bc(negf))
        fasts.append(fast)
        pms.append(pm)
        needs.append((ks[gi] >= 10) & (ks[gi] < negs[gi]))

    need_any = needs[0] | needs[1] | needs[2] | needs[3]
    fast_mine = _sel4(g, *fasts)

    def search_branch():
        k_mine = _sel4(g, *ks)

        def sbody(i, prefix):
            cand = prefix | (jnp.int32(1) << (30 - i))

            def cbody(j, acc):
                x = mbuf[pl.ds(j * L, L)]
                return acc + jnp.where(x >= cand, jnp.int32(1), jnp.int32(0))

            acc = lax.fori_loop(0, E // L, cbody, jnp.zeros((L,), jnp.int32))
            icnt_stage[...] = acc
            pltpu.sync_copy(icnt_stage, icnt_sh.at[pl.ds(s * L, L)])
            plsc.subcore_barrier()
            pltpu.sync_copy(icnt_sh, icnt_rd)
            plsc.subcore_barrier()
            tots = []
            for gi in range(GROUPS):
                tv = jnp.zeros((L,), jnp.int32)
                for wi in range(WPG):
                    tv = tv + icnt_rd[pl.ds((gi * WPG + wi) * L, L)]
                tots.append(jnp.sum(tv))
            tot_mine = _sel4(g, *tots)
            return jnp.where(tot_mine >= k_mine, cand, prefix)

        tau = lax.fori_loop(0, 31, sbody, jnp.int32(0))
        tauf = lax.bitcast_convert_type(tau, jnp.float32)

        def gbody(j, carry):
            accc, accs = carry
            x = mbuf[pl.ds(j * L, L)]
            gt = x > tau
            accc = accc + jnp.where(gt, jnp.int32(1), jnp.int32(0))
            accs = accs + jnp.where(gt, lax.bitcast_convert_type(x, jnp.float32), 0.0)
            return accc, accs

        accc, accs = lax.fori_loop(0, E // L, gbody,
                                   (jnp.zeros((L,), jnp.int32),
                                    jnp.zeros((L,), jnp.float32)))
        icnt_stage[...] = accc
        gsum_stage[...] = accs
        pltpu.sync_copy(icnt_stage, icnt_sh.at[pl.ds(s * L, L)])
        pltpu.sync_copy(gsum_stage, gsum_sh.at[pl.ds(s * L, L)])
        plsc.subcore_barrier()
        pltpu.sync_copy(icnt_sh, icnt_rd)
        pltpu.sync_copy(gsum_sh, gsum_rd)
        cts, sts = [], []
        for gi in range(GROUPS):
            cv = jnp.zeros((L,), jnp.int32)
            sv = jnp.zeros((L,), jnp.float32)
            for wi in range(WPG):
                r = (gi * WPG + wi) * L
                cv = cv + icnt_rd[pl.ds(r, L)]
                sv = sv + gsum_rd[pl.ds(r, L)]
            cts.append(jnp.sum(cv))
            sts.append(jnp.sum(sv))
        cnt_gt = _sel4(g, *cts).astype(jnp.float32)
        sum_gt = _sel4(g, *sts)
        kf = k_mine.astype(jnp.float32)
        topk = sum_gt + (kf - cnt_gt) * tauf
        pm_mine = _sel4(g, *pms)
        need_mine = _sel4(g, *needs)
        return jnp.where(need_mine, pm_mine + bc(topk) / bc(kf), fast_mine)

    loss_mine = lax.cond(need_any, search_branch, lambda: fast_mine)

    @pl.when(s % WPG == 0)
    def _():
        row_buf[...] = loss_mine
        pltpu.sync_copy(row_buf, out_hbm.at[pl.ds((c * GROUPS + g) * L, L)])


_ohem = functools.partial(
    pl.kernel,
    out_type=jax.ShapeDtypeStruct((B * L,), jnp.float32),
    mesh=plsc.VectorSubcoreMesh(core_axis_name="c", subcore_axis_name="s"),
    compiler_params=pltpu.CompilerParams(needs_layout_passes=False),
    scratch_types=[
        pltpu.VMEM((3, CHUNK // 512, 512), jnp.float32),
        pltpu.VMEM((3, CHUNK // 512, 512), jnp.float32),
        pltpu.VMEM((E,), jnp.int32),
        pltpu.VMEM((SROW,), jnp.float32),
        pltpu.VMEM_SHARED((NS * SROW,), jnp.float32),
        pltpu.VMEM((NS * SROW,), jnp.float32),
        pltpu.VMEM((L,), jnp.int32),
        pltpu.VMEM_SHARED((NS * L,), jnp.int32),
        pltpu.VMEM((NS * L,), jnp.int32),
        pltpu.VMEM((L,), jnp.float32),
        pltpu.VMEM_SHARED((NS * L,), jnp.float32),
        pltpu.VMEM((NS * L,), jnp.float32),
        pltpu.VMEM((L,), jnp.float32),
        pltpu.SemaphoreType.DMA,
        pltpu.SemaphoreType.DMA,
        pltpu.SemaphoreType.DMA,
    ],
)(_ohem_body)


def kernel(output_imgs, target_imgs):
    return jnp.mean(_ohem(output_imgs, target_imgs))
